# baseline (speedup 1.0000x reference)
import jax
import jax.numpy as jnp
from jax.experimental import pallas as pl
from jax.experimental.pallas import tpu as pltpu

_X = 48
_H = 64
_B = 8
_NODES = 8
_TD = 100
_NH = 4
_F = 64
_SLOPE = 0.2


def _body(gb):
    n = gb * _B
    rows = n * _NODES

    def body(x_ref, c_ref, t_ref, a_ref, wiou_ref, uiou_ref, ufw_ref,
             wtf_ref, alr_ref, bv_ref, wmlp_ref, tsel_ref, out_ref):
        f32 = jnp.float32
        sig = lambda v: 0.5 * jnp.tanh(0.5 * v) + 0.5
        b_iou = bv_ref[0:1, 0:3 * _H]
        u_f_b = bv_ref[1:2, 0:_H]
        b_mlp = bv_ref[2:3, 0:128]

        x2 = x_ref[...].reshape(rows, _X)
        iou = jnp.dot(x2, wiou_ref[...],
                      preferred_element_type=f32) + b_iou
        io = sig(iou[:, 0:2 * _H])
        u_g = jnp.tanh(iou[:, 2 * _H:3 * _H])
        c_all = io[:, 0:_H] * u_g + c_ref[...].reshape(rows, _H)
        h_all = io[:, _H:2 * _H] * jnp.tanh(c_all)
        f_g = sig(
            jnp.dot(h_all, ufw_ref[...], preferred_element_type=f32) + u_f_b)
        fc = f_g * c_all

        rid = jax.lax.broadcasted_iota(jnp.int32, (rows, 1), 0)
        leaf = (rid % _NODES != 0).astype(f32)
        hc = jnp.concatenate([h_all, fc], axis=1) * leaf
        red = jnp.sum(hc.reshape(n, _NODES, 2 * _H), axis=1)
        h_sum = red[:, 0:_H]
        c_red = red[:, _H:2 * _H]

        iou_r = jnp.dot(h_sum, uiou_ref[...],
                        preferred_element_type=f32) + b_iou
        c_root = (sig(iou_r[:, 0:_H]) *
                  jnp.tanh(iou_r[:, 2 * _H:3 * _H]) + c_red)
        h_root = sig(iou_r[:, _H:2 * _H]) * jnp.tanh(c_root)

        fsrc = jnp.dot(t_ref[...], wtf_ref[...],
                       preferred_element_type=f32)
        cdim = (((1,), (1,)), ((), ()))
        el = jax.lax.dot_general(alr_ref[0:_NH, :], fsrc, cdim,
                                 preferred_element_type=f32)
        er = jax.lax.dot_general(fsrc, alr_ref[_NH:2 * _NH, :], cdim,
                                 preferred_element_type=f32)

        adjm = (a_ref[...] > 0).astype(f32)
        tiled = jnp.dot(adjm, tsel_ref[...],
                        preferred_element_type=f32)
        ri = jax.lax.broadcasted_iota(jnp.int32, (n, n), 0)
        ci = jax.lax.broadcasted_iota(jnp.int32, (n, n), 1)
        mask = jnp.where((ri // _B) == (ci // _B), tiled, 0.0)

        acc = None
        for h in range(_NH):
            e = el[h:h + 1, :] + er[:, h:h + 1]
            e = jnp.maximum(e, _SLOPE * e)
            e = jnp.where(mask > 0, e, -1e30)
            m = jnp.max(e, axis=1, keepdims=True)
            p = jnp.exp(e - m) * mask
            d = jnp.sum(p, axis=1, keepdims=True)
            att = p / jnp.maximum(d, 1e-30)
            r = jnp.dot(att, h_root, preferred_element_type=f32)
            r = jnp.maximum(r + bv_ref[4 + h:5 + h, 0:_F], 0.0)
            acc = r if acc is None else acc + r
        rst_mean = acc * (1.0 / _NH)

        cat = jnp.concatenate([rst_mean, h_root], axis=1)
        out_ref[...] = jnp.dot(cat, wmlp_ref[...],
                               preferred_element_type=f32) + b_mlp

    return body


def kernel(w_iou, b_iou, u_iou, u_f_w, u_f_b, w_type, w_fc, attn_l, attn_r,
           bias_gat, w1, b1, w2, b2, w3, b3, w4, b4,
           x_ast, h0, c0, cfg_type, adj):
    del h0
    f32 = jnp.float32
    g_all = x_ast.shape[0]
    gb = next(d for d in (32, 16, 8, 4, 2, 1) if g_all % d == 0)
    n = gb * _B

    nhf = _NH * _F
    hmask = (jnp.arange(nhf)[None, :] // _F
             == jnp.arange(_NH)[:, None]).astype(f32)
    alr = jnp.concatenate([attn_l * hmask, attn_r * hmask], axis=0)
    wtf = w_type @ w_fc
    wm = w1 @ w2 @ w3 @ w4
    bm = ((b1 @ w2 + b2) @ w3 + b3) @ w4 + b4
    wmlp = jnp.zeros((128, 128), f32).at[:, 0:2].set(wm)
    bvec = (jnp.zeros((8, 256), f32)
            .at[0:1, 0:3 * _H].set(b_iou)
            .at[1:2, 0:_H].set(u_f_b)
            .at[2:3, 0:2].set(bm)
            .at[4:8, 0:_F].set(bias_gat.reshape(_NH, _F)))
    tsel = (jnp.arange(n)[None, :] % _B
            == jnp.arange(_B)[:, None]).astype(f32)

    xb = x_ast.reshape(g_all * _B, _NODES, _X)
    cb = c0.reshape(g_all * _B, _NODES, _H)
    tb = cfg_type.reshape(g_all * _B, _TD)
    ab = adj.reshape(g_all * _B, _B)

    out = pl.pallas_call(
        _body(gb),
        out_shape=jax.ShapeDtypeStruct((g_all * _B, 128), f32),
        grid=(g_all // gb,),
        in_specs=[
            pl.BlockSpec((n, _NODES, _X), lambda g: (g, 0, 0)),
            pl.BlockSpec((n, _NODES, _H), lambda g: (g, 0, 0)),
            pl.BlockSpec((n, _TD), lambda g: (g, 0)),
            pl.BlockSpec((n, _B), lambda g: (g, 0)),
            pl.BlockSpec(w_iou.shape, lambda g: (0, 0)),
            pl.BlockSpec(u_iou.shape, lambda g: (0, 0)),
            pl.BlockSpec(u_f_w.shape, lambda g: (0, 0)),
            pl.BlockSpec((_TD, nhf), lambda g: (0, 0)),
            pl.BlockSpec((2 * _NH, nhf), lambda g: (0, 0)),
            pl.BlockSpec((8, 256), lambda g: (0, 0)),
            pl.BlockSpec((128, 128), lambda g: (0, 0)),
            pl.BlockSpec((_B, n), lambda g: (0, 0)),
        ],
        out_specs=pl.BlockSpec((n, 128), lambda g: (g, 0)),
        compiler_params=pltpu.CompilerParams(
            dimension_semantics=("parallel",)),
        cost_estimate=pl.CostEstimate(
            flops=3_000_000 * g_all, transcendentals=21_000 * g_all,
            bytes_accessed=36_000 * g_all),
    )(xb, cb, tb, ab, w_iou, u_iou, u_f_w, wtf, alr, bvec, wmlp, tsel)

    return out[:, :2].reshape(g_all, _B, 2)


# speedup vs baseline: 1.0364x; 1.0364x over previous
import jax
import jax.numpy as jnp
from jax.experimental import pallas as pl
from jax.experimental.pallas import tpu as pltpu

_X = 48
_H = 64
_B = 8
_NODES = 8
_TD = 100
_NH = 4
_F = 64
_SLOPE = 0.2


def _body(gb):
    n = gb * _B
    rows = n * _NODES

    def body(x_ref, c_ref, t_ref, a_ref, wiou_ref, uiou_ref, ufw_ref,
             wtf_ref, alr_ref, bv_ref, wmlp_ref, tsel_ref, out_ref):
        f32 = jnp.float32
        sig = lambda v: 0.5 * jnp.tanh(0.5 * v) + 0.5
        b_iou = bv_ref[0:1, 0:3 * _H]
        u_f_b = bv_ref[1:2, 0:_H]
        b_mlp = bv_ref[0:8, 255:256]

        x2 = x_ref[...].reshape(rows, _X)
        iou = jnp.dot(x2, wiou_ref[...],
                      preferred_element_type=f32) + b_iou
        io = sig(iou[:, 0:2 * _H])
        u_g = jnp.tanh(iou[:, 2 * _H:3 * _H])
        c_all = io[:, 0:_H] * u_g + c_ref[...].reshape(rows, _H)
        h_all = io[:, _H:2 * _H] * jnp.tanh(c_all)
        f_g = sig(
            jnp.dot(h_all, ufw_ref[...], preferred_element_type=f32) + u_f_b)
        fc = f_g * c_all

        rid = jax.lax.broadcasted_iota(jnp.int32, (rows, 1), 0)
        leaf = (rid % _NODES != 0).astype(f32)
        hc = jnp.concatenate([h_all, fc], axis=1) * leaf
        red = jnp.sum(hc.reshape(n, _NODES, 2 * _H), axis=1)
        h_sum = red[:, 0:_H]
        c_red = red[:, _H:2 * _H]

        iou_r = jnp.dot(h_sum, uiou_ref[...],
                        preferred_element_type=f32) + b_iou
        c_root = (sig(iou_r[:, 0:_H]) *
                  jnp.tanh(iou_r[:, 2 * _H:3 * _H]) + c_red)
        h_root = sig(iou_r[:, _H:2 * _H]) * jnp.tanh(c_root)

        fsrc = jnp.dot(t_ref[...].reshape(n, _TD), wtf_ref[...],
                       preferred_element_type=f32)
        cdim = (((1,), (1,)), ((), ()))
        el = jax.lax.dot_general(alr_ref[0:_NH, :], fsrc, cdim,
                                 preferred_element_type=f32)
        er = jax.lax.dot_general(fsrc, alr_ref[_NH:2 * _NH, :], cdim,
                                 preferred_element_type=f32)

        adjm = (a_ref[...].reshape(n, _B) > 0).astype(f32)
        tiled = jnp.dot(adjm, tsel_ref[...],
                        preferred_element_type=f32)
        ri = jax.lax.broadcasted_iota(jnp.int32, (n, n), 0)
        ci = jax.lax.broadcasted_iota(jnp.int32, (n, n), 1)
        mask = jnp.where((ri // _B) == (ci // _B), tiled, 0.0)

        acc = None
        for h in range(_NH):
            e = el[h:h + 1, :] + er[:, h:h + 1]
            e = jnp.maximum(e, _SLOPE * e)
            e = jnp.where(mask > 0, e, -1e30)
            m = jnp.max(e, axis=1, keepdims=True)
            p = jnp.exp(e - m) * mask
            d = jnp.sum(p, axis=1, keepdims=True)
            att = p / jnp.maximum(d, 1e-30)
            r = jnp.dot(att, h_root, preferred_element_type=f32)
            r = jnp.maximum(r + bv_ref[4 + h:5 + h, 0:_F], 0.0)
            acc = r if acc is None else acc + r
        rst_mean = acc * (1.0 / _NH)

        cat = jnp.concatenate([rst_mean, h_root], axis=1)
        out_ref[...] = jax.lax.dot_general(
            wmlp_ref[...], cat, (((1,), (1,)), ((), ())),
            preferred_element_type=f32) + b_mlp

    return body


def kernel(w_iou, b_iou, u_iou, u_f_w, u_f_b, w_type, w_fc, attn_l, attn_r,
           bias_gat, w1, b1, w2, b2, w3, b3, w4, b4,
           x_ast, h0, c0, cfg_type, adj):
    del h0
    f32 = jnp.float32
    g_all = x_ast.shape[0]
    gb = next(d for d in (32, 16, 8, 4, 2, 1) if g_all % d == 0)
    n = gb * _B

    nhf = _NH * _F
    hmask = (jnp.arange(nhf)[None, :] // _F
             == jnp.arange(_NH)[:, None]).astype(f32)
    alr = jnp.concatenate([attn_l * hmask, attn_r * hmask], axis=0)
    wtf = w_type @ w_fc
    wm = w1 @ w2 @ w3 @ w4
    bm = ((b1 @ w2 + b2) @ w3 + b3) @ w4 + b4
    wmlp = jnp.zeros((8, 128), f32).at[0:2, :].set(wm.T)
    bvec = (jnp.zeros((8, 256), f32)
            .at[0:1, 0:3 * _H].set(b_iou)
            .at[1:2, 0:_H].set(u_f_b)
            .at[0:2, 255:256].set(bm.T)
            .at[4:8, 0:_F].set(bias_gat.reshape(_NH, _F)))
    tsel = (jnp.arange(n)[None, :] % _B
            == jnp.arange(_B)[:, None]).astype(f32)

    xb = x_ast.reshape(g_all * _B, _NODES, _X)
    cb = c0.reshape(g_all * _B, _NODES, _H)

    out = pl.pallas_call(
        _body(gb),
        out_shape=jax.ShapeDtypeStruct((8, g_all * _B), f32),
        grid=(g_all // gb,),
        in_specs=[
            pl.BlockSpec((n, _NODES, _X), lambda g: (g, 0, 0)),
            pl.BlockSpec((n, _NODES, _H), lambda g: (g, 0, 0)),
            pl.BlockSpec((gb, _B, _TD), lambda g: (g, 0, 0)),
            pl.BlockSpec((gb, _B, _B), lambda g: (g, 0, 0)),
            pl.BlockSpec(w_iou.shape, lambda g: (0, 0)),
            pl.BlockSpec(u_iou.shape, lambda g: (0, 0)),
            pl.BlockSpec(u_f_w.shape, lambda g: (0, 0)),
            pl.BlockSpec((_TD, nhf), lambda g: (0, 0)),
            pl.BlockSpec((2 * _NH, nhf), lambda g: (0, 0)),
            pl.BlockSpec((8, 256), lambda g: (0, 0)),
            pl.BlockSpec((8, 128), lambda g: (0, 0)),
            pl.BlockSpec((_B, n), lambda g: (0, 0)),
        ],
        out_specs=pl.BlockSpec((8, n), lambda g: (0, g)),
        compiler_params=pltpu.CompilerParams(
            dimension_semantics=("parallel",)),
        cost_estimate=pl.CostEstimate(
            flops=3_000_000 * g_all, transcendentals=21_000 * g_all,
            bytes_accessed=36_000 * g_all),
    )(xb, cb, cfg_type, adj, w_iou, u_iou, u_f_w, wtf, alr, bvec, wmlp, tsel)

    return jnp.transpose(out[0:2, :]).reshape(g_all, _B, 2)


# speedup vs baseline: 1.2226x; 1.1797x over previous
import jax
import jax.numpy as jnp
from jax.experimental import pallas as pl
from jax.experimental.pallas import tpu as pltpu

_X = 48
_H = 64
_B = 8
_NODES = 8
_TD = 100
_NH = 4
_F = 64
_SLOPE = 0.2
_GL = 128
_CL = 32


def _body(x_ref, c_ref, t_ref, a_ref, wiou_ref, uiou_ref, ufw_ref,
          wtf_ref, alr_ref, bv_ref, wmlp_ref, tsel_ref, out_ref):
    f32 = jnp.float32
    sig = lambda v: 0.5 * jnp.tanh(0.5 * v) + 0.5
    b_iou = bv_ref[0:1, 0:3 * _H]
    u_f_b = bv_ref[1:2, 0:_H]
    b_mlp = bv_ref[0:8, 255:256]
    rows = _B * _NODES * _GL
    n_root = _B * _GL

    x2 = jnp.transpose(x_ref[...], (0, 1, 3, 2)).reshape(rows, _X)
    c2 = jnp.transpose(c_ref[...], (0, 1, 3, 2)).reshape(rows, _H)

    iou = jnp.dot(x2, wiou_ref[...],
                  preferred_element_type=f32) + b_iou
    io = sig(iou[:, 0:2 * _H])
    u_g = jnp.tanh(iou[:, 2 * _H:3 * _H])
    c_all = io[:, 0:_H] * u_g + c2
    h_all = io[:, _H:2 * _H] * jnp.tanh(c_all)
    f_g = sig(
        jnp.dot(h_all, ufw_ref[...], preferred_element_type=f32) + u_f_b)
    fc = f_g * c_all

    rid = jax.lax.broadcasted_iota(jnp.int32, (rows, 1), 0)
    leaf = ((rid // _GL) % _NODES != 0).astype(f32)
    hc = jnp.concatenate([h_all, fc], axis=1) * leaf
    red = jnp.sum(hc.reshape(_B, _NODES, _GL, 2 * _H), axis=1)
    red = red.reshape(n_root, 2 * _H)
    h_sum = red[:, 0:_H]
    c_red = red[:, _H:2 * _H]

    iou_r = jnp.dot(h_sum, uiou_ref[...],
                    preferred_element_type=f32) + b_iou
    c_root = (sig(iou_r[:, 0:_H]) *
              jnp.tanh(iou_r[:, 2 * _H:3 * _H]) + c_red)
    h_root = sig(iou_r[:, _H:2 * _H]) * jnp.tanh(c_root)

    t3 = t_ref[...]
    t_n = jnp.concatenate(
        [jnp.transpose(t3[:, b, :], (1, 0)) for b in range(_B)],
        axis=0)
    fsrc = jnp.dot(t_n, wtf_ref[...],
                   preferred_element_type=f32)
    a3 = a_ref[...]
    adj_n = jnp.concatenate(
        [jnp.transpose(a3[d], (1, 0)) for d in range(_B)],
        axis=0)

    nc = _B * _CL
    h_root3 = h_root.reshape(_B, _GL, _H)
    fsrc3 = fsrc.reshape(_B, _GL, 2 * _H * 2)
    adj3 = adj_n.reshape(_B, _GL, _B)
    cdim = (((1,), (1,)), ((), ()))
    ri = jax.lax.broadcasted_iota(jnp.int32, (nc, nc), 0)
    ci = jax.lax.broadcasted_iota(jnp.int32, (nc, nc), 1)
    sameg = (ri % _CL) == (ci % _CL)
    outs = []
    for cx in range(_GL // _CL):
        sl = slice(_CL * cx, _CL * (cx + 1))
        hr = h_root3[:, sl, :].reshape(nc, _H)
        fs = fsrc3[:, sl, :].reshape(nc, 2 * _H * 2)
        am = (adj3[:, sl, :].reshape(nc, _B) > 0).astype(f32)
        el = jax.lax.dot_general(alr_ref[0:_NH, :], fs, cdim,
                                 preferred_element_type=f32)
        er = jax.lax.dot_general(fs, alr_ref[_NH:2 * _NH, :], cdim,
                                 preferred_element_type=f32)
        tiled = jnp.dot(am, tsel_ref[...],
                        preferred_element_type=f32)
        mask = jnp.where(sameg, tiled, 0.0)
        acc = None
        for h in range(_NH):
            e = el[h:h + 1, :] + er[:, h:h + 1]
            e = jnp.maximum(e, _SLOPE * e)
            e = jnp.where(mask > 0, e, -1e30)
            m = jnp.max(e, axis=1, keepdims=True)
            p = jnp.exp(e - m) * mask
            d = jnp.sum(p, axis=1, keepdims=True)
            att = p / jnp.maximum(d, 1e-30)
            r = jnp.dot(att, hr, preferred_element_type=f32)
            r = jnp.maximum(r + bv_ref[4 + h:5 + h, 0:_F], 0.0)
            acc = r if acc is None else acc + r
        cat = jnp.concatenate([acc * (1.0 / _NH), hr], axis=1)
        outs.append(jax.lax.dot_general(
            wmlp_ref[...], cat, cdim,
            preferred_element_type=f32) + b_mlp)
    out_ref[...] = jnp.concatenate(outs, axis=1)


def kernel(w_iou, b_iou, u_iou, u_f_w, u_f_b, w_type, w_fc, attn_l, attn_r,
           bias_gat, w1, b1, w2, b2, w3, b3, w4, b4,
           x_ast, h0, c0, cfg_type, adj):
    del h0
    f32 = jnp.float32
    g_all = x_ast.shape[0]
    steps = g_all // _GL
    nc = _B * _CL

    nhf = _NH * _F
    hmask = (jnp.arange(nhf)[None, :] // _F
             == jnp.arange(_NH)[:, None]).astype(f32)
    alr = jnp.concatenate([attn_l * hmask, attn_r * hmask], axis=0)
    wtf = w_type @ w_fc
    wm = w1 @ w2 @ w3 @ w4
    bm = ((b1 @ w2 + b2) @ w3 + b3) @ w4 + b4
    wmlp = jnp.zeros((8, 128), f32).at[0:2, :].set(wm.T)
    bvec = (jnp.zeros((8, 256), f32)
            .at[0:1, 0:3 * _H].set(b_iou)
            .at[1:2, 0:_H].set(u_f_b)
            .at[0:2, 255:256].set(bm.T)
            .at[4:8, 0:_F].set(bias_gat.reshape(_NH, _F)))
    tsel = (jnp.arange(nc)[None, :] // _CL
            == jnp.arange(_B)[:, None]).astype(f32)

    hbm = lambda v: pltpu.with_memory_space_constraint(
        v, pltpu.MemorySpace.HBM)
    xb = hbm(jnp.transpose(x_ast, (1, 2, 3, 0)))
    cb = hbm(jnp.transpose(c0, (1, 2, 3, 0)))
    tb = hbm(jnp.transpose(cfg_type, (2, 1, 0)))
    ab = hbm(jnp.transpose(adj, (1, 2, 0)))

    out = pl.pallas_call(
        _body,
        out_shape=jax.ShapeDtypeStruct((8, g_all * _B), f32),
        grid=(steps,),
        in_specs=[
            pl.BlockSpec((_B, _NODES, _X, _GL), lambda g: (0, 0, 0, g)),
            pl.BlockSpec((_B, _NODES, _H, _GL), lambda g: (0, 0, 0, g)),
            pl.BlockSpec((_TD, _B, _GL), lambda g: (0, 0, g)),
            pl.BlockSpec((_B, _B, _GL), lambda g: (0, 0, g)),
            pl.BlockSpec(w_iou.shape, lambda g: (0, 0)),
            pl.BlockSpec(u_iou.shape, lambda g: (0, 0)),
            pl.BlockSpec(u_f_w.shape, lambda g: (0, 0)),
            pl.BlockSpec((_TD, nhf), lambda g: (0, 0)),
            pl.BlockSpec((2 * _NH, nhf), lambda g: (0, 0)),
            pl.BlockSpec((8, 256), lambda g: (0, 0)),
            pl.BlockSpec((8, 128), lambda g: (0, 0)),
            pl.BlockSpec((_B, nc), lambda g: (0, 0)),
        ],
        out_specs=pl.BlockSpec((8, _B * _GL), lambda g: (0, g)),
        compiler_params=pltpu.CompilerParams(
            dimension_semantics=("parallel",)),
        cost_estimate=pl.CostEstimate(
            flops=3_000_000 * g_all, transcendentals=21_000 * g_all,
            bytes_accessed=36_000 * g_all),
    )(xb, cb, tb, ab, w_iou, u_iou, u_f_w, wtf, alr, bvec, wmlp, tsel)

    o = out.reshape(8, steps, _GL // _CL, _B, _CL)
    o = jnp.transpose(o, (1, 2, 4, 3, 0))
    return o.reshape(g_all, _B, 8)[:, :, 0:2]
